# Initial kernel scaffold; baseline (speedup 1.0000x reference)
import jax
import jax.numpy as jnp
from jax.experimental import pallas as pl


def kernel(x, pos, W0_0, b0_0, g0_0, be0_0, W0_1, b0_1, g0_1, be0_1, W0_2, b0_2, g0_2, be0_2, W1_0, b1_0, g1_0, be1_0, W1_1, b1_1, g1_1, be1_1, W1_2, b1_2, g1_2, be1_2):
    raise NotImplementedError("write your pallas kernel here")


# speedup vs baseline: 5.8370x; 5.8370x over previous
import jax
import jax.numpy as jnp
from jax.experimental import pallas as pl
from jax.experimental.pallas import tpu as pltpu

_S = 128
_K = 64
_RADIUS = 0.4
_NEG = -100000000.0


def _fps_kernel(pos_ref, out_ref):
    B, _, N = pos_ref.shape
    px = pos_ref[:, 0, :]
    py = pos_ref[:, 1, :]
    pz = pos_ref[:, 2, :]
    lane = jax.lax.broadcasted_iota(jnp.int32, (B, N), 1)
    slane = jax.lax.broadcasted_iota(jnp.int32, (B, _S), 1)

    lx = px[:, 0:1]
    ly = py[:, 0:1]
    lz = pz[:, 0:1]
    sx = jnp.where(slane == 0, lx, 0.0)
    sy = jnp.where(slane == 0, ly, 0.0)
    sz = jnp.where(slane == 0, lz, 0.0)
    mind = jnp.full((B, N), 1e10, jnp.float32)

    def body(i, c):
        mind, lx, ly, lz, sx, sy, sz = c
        dx = px - lx
        dy = py - ly
        dz = pz - lz
        d = (dx * dx + dy * dy) + dz * dz
        mind = jnp.minimum(mind, d)
        m = jnp.max(mind, axis=1, keepdims=True)
        idx = jnp.min(jnp.where(mind == m, lane, N), axis=1, keepdims=True)
        sel = lane == idx
        lx = jnp.sum(jnp.where(sel, px, 0.0), axis=1, keepdims=True)
        ly = jnp.sum(jnp.where(sel, py, 0.0), axis=1, keepdims=True)
        lz = jnp.sum(jnp.where(sel, pz, 0.0), axis=1, keepdims=True)
        sx = jnp.where(slane == i, lx, sx)
        sy = jnp.where(slane == i, ly, sy)
        sz = jnp.where(slane == i, lz, sz)
        return mind, lx, ly, lz, sx, sy, sz

    _, _, _, _, sx, sy, sz = jax.lax.fori_loop(
        1, _S, body, (mind, lx, ly, lz, sx, sy, sz))
    out_ref[:, 0, :] = sx
    out_ref[:, 1, :] = sy
    out_ref[:, 2, :] = sz


def _fps(pos_b3n):
    B, _, N = pos_b3n.shape
    return pl.pallas_call(
        _fps_kernel,
        out_shape=jax.ShapeDtypeStruct((B, 3, _S), jnp.float32),
    )(pos_b3n)


def _knn_kernel(pos_ref, smp_ref, idx_ref, dsel_ref, dm_ref):
    N = pos_ref.shape[2]
    px = pos_ref[0, 0, :][None, :]
    py = pos_ref[0, 1, :][None, :]
    pz = pos_ref[0, 2, :][None, :]
    sx = smp_ref[0, 0, :][:, None]
    sy = smp_ref[0, 1, :][:, None]
    sz = smp_ref[0, 2, :][:, None]
    dx = sx - px
    dy = sy - py
    dz = sz - pz
    dm_ref[...] = jnp.sqrt((dx * dx + dy * dy) + dz * dz)

    lane = jax.lax.broadcasted_iota(jnp.int32, (_S, N), 1)
    kiota = jax.lax.broadcasted_iota(jnp.int32, (_S, _K), 1)

    def body(j, c):
        idxs, dsel = c
        cur = dm_ref[...]
        m = jnp.min(cur, axis=1, keepdims=True)
        idx = jnp.min(jnp.where(cur == m, lane, N), axis=1, keepdims=True)
        dm_ref[...] = jnp.where(lane == idx, jnp.inf, cur)
        idxs = jnp.where(kiota == j, idx, idxs)
        dsel = jnp.where(kiota == j, m, dsel)
        return idxs, dsel

    idxs, dsel = jax.lax.fori_loop(
        0, _K, body,
        (jnp.zeros((_S, _K), jnp.int32), jnp.zeros((_S, _K), jnp.float32)))
    idx_ref[0] = idxs
    dsel_ref[0] = dsel


def _knn(pos_b3n, smp_b3s):
    B, _, N = pos_b3n.shape
    return pl.pallas_call(
        _knn_kernel,
        grid=(B,),
        in_specs=[
            pl.BlockSpec((1, 3, N), lambda b: (b, 0, 0)),
            pl.BlockSpec((1, 3, _S), lambda b: (b, 0, 0)),
        ],
        out_specs=[
            pl.BlockSpec((1, _S, _K), lambda b: (b, 0, 0)),
            pl.BlockSpec((1, _S, _K), lambda b: (b, 0, 0)),
        ],
        out_shape=[
            jax.ShapeDtypeStruct((B, _S, _K), jnp.int32),
            jax.ShapeDtypeStruct((B, _S, _K), jnp.float32),
        ],
        scratch_shapes=[pltpu.VMEM((_S, N), jnp.float32)],
        compiler_params=pltpu.CompilerParams(
            dimension_semantics=("arbitrary",)),
    )(pos_b3n, smp_b3s)


def _mlp0_p1_kernel(g_ref, crep_ref, w_ref, b_ref, feat_ref, stats_ref,
                    acc_ref):
    b = pl.program_id(0)
    nb = pl.num_programs(0)
    g = g_ref[0]
    gp = g[:, 0:3] - crep_ref[0]
    in6 = jnp.concatenate([gp, g[:, 3:6]], axis=1)
    f = jnp.dot(in6, w_ref[0:6, :], preferred_element_type=jnp.float32)
    f = f + b_ref[0:1, :]
    feat_ref[0] = f

    @pl.when(b == 0)
    def _():
        acc_ref[...] = jnp.zeros_like(acc_ref)

    s1 = jnp.sum(f, axis=0, keepdims=True)
    s2 = jnp.sum(f * f, axis=0, keepdims=True)
    acc_ref[0:1, :] = acc_ref[0:1, :] + s1
    acc_ref[1:2, :] = acc_ref[1:2, :] + s2

    @pl.when(b == nb - 1)
    def _():
        stats_ref[...] = acc_ref[...]


def _mlp0_mid_kernel(f_ref, sc_ref, w_ref, b_ref, feat_ref, stats_ref,
                     acc_ref):
    b = pl.program_id(0)
    nb = pl.num_programs(0)
    h = jnp.maximum(f_ref[0] * sc_ref[0:1, :] + sc_ref[1:2, :], 0.0)
    f = jnp.dot(h, w_ref[...], preferred_element_type=jnp.float32)
    f = f + b_ref[0:1, :]
    feat_ref[0] = f

    @pl.when(b == 0)
    def _():
        acc_ref[...] = jnp.zeros_like(acc_ref)

    s1 = jnp.sum(f, axis=0, keepdims=True)
    s2 = jnp.sum(f * f, axis=0, keepdims=True)
    acc_ref[0:1, :] = acc_ref[0:1, :] + s1
    acc_ref[1:2, :] = acc_ref[1:2, :] + s2

    @pl.when(b == nb - 1)
    def _():
        stats_ref[...] = acc_ref[...]


def _mlp0_p4_kernel(f_ref, sc_ref, dsel_ref, x1_ref):
    h = jnp.maximum(f_ref[0] * sc_ref[0:1, :] + sc_ref[1:2, :], 0.0)
    mask = dsel_ref[0, 0, :][:, None] <= _RADIUS
    h = jnp.where(mask, h, _NEG)
    h3 = jnp.reshape(h, (_S, _K, h.shape[1]))
    x1_ref[0] = jnp.max(h3, axis=1)


def _mlp0_p1(grouped, crep, w, bvec):
    B = grouped.shape[0]
    C = w.shape[1]
    M = _S * _K
    wp = jnp.zeros((8, C), jnp.float32).at[0:6, :].set(w)
    bp = jnp.broadcast_to(bvec[None, :], (8, C))
    return pl.pallas_call(
        _mlp0_p1_kernel,
        grid=(B,),
        in_specs=[
            pl.BlockSpec((1, M, 6), lambda b: (b, 0, 0)),
            pl.BlockSpec((1, M, 3), lambda b: (b, 0, 0)),
            pl.BlockSpec((8, C), lambda b: (0, 0)),
            pl.BlockSpec((8, C), lambda b: (0, 0)),
        ],
        out_specs=[
            pl.BlockSpec((1, M, C), lambda b: (b, 0, 0)),
            pl.BlockSpec((8, C), lambda b: (0, 0)),
        ],
        out_shape=[
            jax.ShapeDtypeStruct((B, M, C), jnp.float32),
            jax.ShapeDtypeStruct((8, C), jnp.float32),
        ],
        scratch_shapes=[pltpu.VMEM((8, C), jnp.float32)],
        compiler_params=pltpu.CompilerParams(
            dimension_semantics=("arbitrary",)),
    )(grouped.reshape(B, M, 6), crep, wp, bp)


def _mlp0_mid(feat, scale, shift, w, bvec):
    B, M, Cin = feat.shape
    C = w.shape[1]
    sc = jnp.zeros((8, Cin), jnp.float32)
    sc = sc.at[0, :].set(scale).at[1, :].set(shift)
    bp = jnp.broadcast_to(bvec[None, :], (8, C))
    return pl.pallas_call(
        _mlp0_mid_kernel,
        grid=(B,),
        in_specs=[
            pl.BlockSpec((1, M, Cin), lambda b: (b, 0, 0)),
            pl.BlockSpec((8, Cin), lambda b: (0, 0)),
            pl.BlockSpec((Cin, C), lambda b: (0, 0)),
            pl.BlockSpec((8, C), lambda b: (0, 0)),
        ],
        out_specs=[
            pl.BlockSpec((1, M, C), lambda b: (b, 0, 0)),
            pl.BlockSpec((8, C), lambda b: (0, 0)),
        ],
        out_shape=[
            jax.ShapeDtypeStruct((B, M, C), jnp.float32),
            jax.ShapeDtypeStruct((8, C), jnp.float32),
        ],
        scratch_shapes=[pltpu.VMEM((8, C), jnp.float32)],
        compiler_params=pltpu.CompilerParams(
            dimension_semantics=("arbitrary",)),
    )(feat, sc, w, bp)


def _mlp0_p4(feat, scale, shift, dsel):
    B, M, C = feat.shape
    sc = jnp.zeros((8, C), jnp.float32)
    sc = sc.at[0, :].set(scale).at[1, :].set(shift)
    return pl.pallas_call(
        _mlp0_p4_kernel,
        grid=(B,),
        in_specs=[
            pl.BlockSpec((1, M, C), lambda b: (b, 0, 0)),
            pl.BlockSpec((8, C), lambda b: (0, 0)),
            pl.BlockSpec((1, 1, M), lambda b: (b, 0, 0)),
        ],
        out_specs=pl.BlockSpec((1, _S, C), lambda b: (b, 0, 0)),
        out_shape=jax.ShapeDtypeStruct((B, _S, C), jnp.float32),
        compiler_params=pltpu.CompilerParams(
            dimension_semantics=("arbitrary",)),
    )(feat, sc, dsel.reshape(B, 1, M))


def _bn_affine(stats, g, be, count):
    s1 = stats[0]
    s2 = stats[1]
    m = s1 / count
    v = s2 / count - m * m
    scale = g / jnp.sqrt(v + 1e-5)
    shift = be - m * scale
    return scale, shift


def _mlp1_kernel(x1_ref, p_ref, w0_ref, w1_ref, w2_ref, bg_ref, out_ref):
    B = x1_ref.shape[0]
    M = B * _S
    x1 = jnp.reshape(x1_ref[...], (M, x1_ref.shape[2]))
    p = jnp.reshape(p_ref[...], (M, 3))
    feat = jnp.concatenate([p, x1], axis=1)
    for li, w_ref in enumerate((w0_ref, w1_ref, w2_ref)):
        C = w_ref.shape[1]
        f = jnp.dot(feat, w_ref[...], preferred_element_type=jnp.float32)
        f = f + bg_ref[4 * li + 0, 0:1, :C]
        m = jnp.mean(f, axis=0, keepdims=True)
        v = jnp.mean(jnp.square(f - m), axis=0, keepdims=True)
        f = (f - m) / jnp.sqrt(v + 1e-5)
        f = f * bg_ref[4 * li + 1, 0:1, :C] + bg_ref[4 * li + 2, 0:1, :C]
        feat = jnp.maximum(f, 0.0)
    gf = jnp.reshape(feat, (B, _S, feat.shape[1]))
    out_ref[...] = jnp.max(gf, axis=1)


def _mlp1(x1, pos_rows, params1):
    B = x1.shape[0]
    Cout = params1[2][0].shape[1]
    bg = jnp.zeros((12, 8, Cout), jnp.float32)
    for li, (w, bvec, g, be) in enumerate(params1):
        C = w.shape[1]
        bg = bg.at[4 * li + 0, 0, :C].set(bvec)
        bg = bg.at[4 * li + 1, 0, :C].set(g)
        bg = bg.at[4 * li + 2, 0, :C].set(be)
    w0p = jnp.zeros((136, params1[0][0].shape[1]), jnp.float32)
    w0p = w0p.at[0:131, :].set(params1[0][0])
    return pl.pallas_call(
        _mlp1_kernel,
        out_shape=jax.ShapeDtypeStruct((B, Cout), jnp.float32),
    )(x1, pos_rows, w0p[0:131], params1[1][0], params1[2][0], bg)


def kernel(x, pos, W0_0, b0_0, g0_0, be0_0, W0_1, b0_1, g0_1, be0_1,
           W0_2, b0_2, g0_2, be0_2, W1_0, b1_0, g1_0, be1_0,
           W1_1, b1_1, g1_1, be1_1, W1_2, b1_2, g1_2, be1_2):
    B, N, _ = pos.shape
    pos_b3n = jnp.transpose(pos, (0, 2, 1))

    smp = _fps(pos_b3n)
    topk_idx, dsel = _knn(pos_b3n, smp)

    table = jnp.concatenate([pos, x], axis=-1)
    flat_idx = topk_idx.reshape(B, _S * _K, 1)
    grouped = jnp.take_along_axis(table, flat_idx, axis=1)

    pos_rows = jnp.transpose(smp, (0, 2, 1))
    crep = jnp.repeat(pos_rows, _K, axis=1)

    cnt = float(B * _S * _K)
    f0, st0 = _mlp0_p1(grouped, crep, W0_0, b0_0)
    sc0, sh0 = _bn_affine(st0, g0_0, be0_0, cnt)
    f1, st1 = _mlp0_mid(f0, sc0, sh0, W0_1, b0_1)
    sc1, sh1 = _bn_affine(st1, g0_1, be0_1, cnt)
    f2, st2 = _mlp0_mid(f1, sc1, sh1, W0_2, b0_2)
    sc2, sh2 = _bn_affine(st2, g0_2, be0_2, cnt)
    x1 = _mlp0_p4(f2, sc2, sh2, dsel)

    params1 = [(W1_0, b1_0, g1_0, be1_0), (W1_1, b1_1, g1_1, be1_1),
               (W1_2, b1_2, g1_2, be1_2)]
    gx = _mlp1(x1, pos_rows, params1)

    global_x = gx[:, None, :]
    pos_out = jnp.zeros((B, 1, 3), jnp.float32)
    return global_x, pos_out
